# baseline (speedup 1.0000x reference)
import functools

import jax
import jax.numpy as jnp
from jax import lax
from jax.experimental import pallas as pl
from jax.experimental.pallas import tpu as pltpu
from jax.experimental.pallas import tpu_sc as plsc

_NBUF = 3


def kernel(spikes, embed):
    b, t, c, h = spikes.shape
    d = embed.shape[-1]
    n = b * t * c * h
    idx = spikes.reshape(n)

    info = plsc.get_sparse_core_info()
    nw = info.num_cores * info.num_subcores
    per_w = n // nw
    chunk = 1024
    n_chunks = per_w // chunk

    mesh = plsc.VectorSubcoreMesh(core_axis_name="c", subcore_axis_name="s")

    @functools.partial(
        pl.kernel,
        mesh=mesh,
        compiler_params=pltpu.CompilerParams(use_tc_tiling_on_sc=False),
        out_type=jax.ShapeDtypeStruct((n, d), jnp.float32),
        scratch_types=[
            pltpu.VMEM((_NBUF, chunk), jnp.int32),
            pltpu.VMEM((_NBUF, chunk, d), jnp.float32),
            pltpu.SemaphoreType.DMA((_NBUF,)),
            pltpu.SemaphoreType.DMA((_NBUF,)),
        ],
    )
    def run(idx_hbm, table_hbm, out_hbm, idx_v, rows_v, gsem, osem):
        wid = lax.axis_index("s") * info.num_cores + lax.axis_index("c")
        base = wid * per_w

        def start_gather(i):
            s = i % _NBUF
            pltpu.sync_copy(idx_hbm.at[pl.ds(base + i * chunk, chunk)],
                            idx_v.at[s])
            return pltpu.async_copy(table_hbm.at[idx_v.at[s]], rows_v.at[s],
                                    gsem.at[s])

        def start_out(i):
            s = i % _NBUF
            return pltpu.async_copy(rows_v.at[s],
                                    out_hbm.at[pl.ds(base + i * chunk, chunk)],
                                    osem.at[s])

        gathers = {}
        outs = {}
        for i in range(n_chunks):
            if i >= _NBUF:
                outs.pop(i - _NBUF).wait()
            gathers[i] = start_gather(i)
            if i >= 1:
                gathers.pop(i - 1).wait()
                outs[i - 1] = start_out(i - 1)
        gathers.pop(n_chunks - 1).wait()
        outs[n_chunks - 1] = start_out(n_chunks - 1)
        for i in sorted(outs):
            outs.pop(i).wait()

    out = run(idx, embed)
    return out.reshape(b, t, c * h * d)


# speedup vs baseline: 3.8253x; 3.8253x over previous
import functools

import jax
import jax.numpy as jnp
from jax import lax
from jax.experimental import pallas as pl
from jax.experimental.pallas import tpu as pltpu
from jax.experimental.pallas import tpu_sc as plsc

_NBUF = 3
_UNROLL = 8


def kernel(spikes, embed):
    b, t, c, h = spikes.shape
    v_rows, d = embed.shape
    n = b * t * c * h
    idx = spikes.reshape(n)

    info = plsc.get_sparse_core_info()
    nw = info.num_cores * info.num_subcores
    lanes = info.num_lanes
    per_w = n // nw
    chunk = 1024
    n_chunks = per_w // chunk

    mesh = plsc.VectorSubcoreMesh(core_axis_name="c", subcore_axis_name="s")

    @functools.partial(
        pl.kernel,
        mesh=mesh,
        out_type=jax.ShapeDtypeStruct((n * d,), jnp.float32),
        scratch_types=[
            pltpu.VMEM((v_rows * d,), jnp.float32),
            pltpu.VMEM((_NBUF * chunk,), jnp.int32),
            pltpu.VMEM((_NBUF * chunk * d,), jnp.float32),
            pltpu.SemaphoreType.DMA((_NBUF,)),
            pltpu.SemaphoreType.DMA((_NBUF,)),
        ],
    )
    def run(idx_hbm, table_hbm, out_hbm, table_v, idx_v, rows_v, isem, osem):
        wid = lax.axis_index("s") * info.num_cores + lax.axis_index("c")
        base = wid * per_w

        pltpu.sync_copy(table_hbm, table_v)

        def idx_copy(i, s):
            return pltpu.make_async_copy(
                idx_hbm.at[pl.ds(base + i * chunk, chunk)],
                idx_v.at[pl.ds(s * chunk, chunk)], isem.at[s])

        def out_copy(i, s):
            return pltpu.make_async_copy(
                rows_v.at[pl.ds(s * chunk * d, chunk * d)],
                out_hbm.at[pl.ds((base + i * chunk) * d, chunk * d)],
                osem.at[s])

        def compute(s):
            def kbody(k, carry):
                ivec = idx_v[pl.ds((s * chunk + k * lanes), lanes)]
                for u in range(lanes):
                    off = ivec[u] * d
                    obase = (s * chunk + k * lanes + u) * d
                    for half in range(d // lanes):
                        rows_v[pl.ds(obase + half * lanes, lanes)] = (
                            table_v[pl.ds(off + half * lanes, lanes)])
                return carry

            lax.fori_loop(0, chunk // lanes, kbody, 0)

        idx_copy(0, 0).start()

        def chunk_body(i, carry):
            s = lax.rem(i, _NBUF)

            @pl.when(i + 1 < n_chunks)
            def _():
                idx_copy(i + 1, lax.rem(i + 1, _NBUF)).start()

            idx_copy(i, s).wait()

            @pl.when(i >= _NBUF)
            def _():
                out_copy(i - _NBUF, s).wait()

            compute(s)
            out_copy(i, s).start()
            return carry

        lax.fori_loop(0, n_chunks, chunk_body, 0)
        for j in range(_NBUF):
            i = n_chunks - _NBUF + j
            out_copy(i, i % _NBUF).wait()

    out = run(idx, embed.reshape(v_rows * d))
    return out.reshape(b, t, c * h * d)


# speedup vs baseline: 7.1314x; 1.8643x over previous
import functools

import jax
import jax.numpy as jnp
from jax import lax
from jax.experimental import pallas as pl
from jax.experimental.pallas import tpu as pltpu
from jax.experimental.pallas import tpu_sc as plsc

_NBUF = 3
_UNROLL = 8


def kernel(spikes, embed):
    b, t, c, h = spikes.shape
    v_rows, d = embed.shape
    n = b * t * c * h
    idx = spikes.reshape(n)

    info = plsc.get_sparse_core_info()
    nw = info.num_cores * info.num_subcores
    lanes = info.num_lanes
    per_w = n // nw
    chunk = 1024
    n_chunks = per_w // chunk

    mesh = plsc.VectorSubcoreMesh(core_axis_name="c", subcore_axis_name="s")

    @functools.partial(
        pl.kernel,
        mesh=mesh,
        out_type=jax.ShapeDtypeStruct((n * d,), jnp.float32),
        scratch_types=[
            pltpu.VMEM((v_rows * d,), jnp.float32),
            pltpu.VMEM((_NBUF * chunk,), jnp.int32),
            pltpu.VMEM((_NBUF * chunk * d,), jnp.float32),
            pltpu.SemaphoreType.DMA((_NBUF,)),
            pltpu.SemaphoreType.DMA((_NBUF,)),
        ],
    )
    def run(idx_hbm, table_hbm, out_hbm, table_v, idx_v, rows_v, isem, osem):
        wid = lax.axis_index("s") * info.num_cores + lax.axis_index("c")
        base = wid * per_w

        pltpu.sync_copy(table_hbm, table_v)

        def idx_copy(i, s):
            return pltpu.make_async_copy(
                idx_hbm.at[pl.ds(base + i * chunk, chunk)],
                idx_v.at[pl.ds(s * chunk, chunk)], isem.at[s])

        def out_copy(i, s):
            return pltpu.make_async_copy(
                rows_v.at[pl.ds(s * chunk * d, chunk * d)],
                out_hbm.at[pl.ds((base + i * chunk) * d, chunk * d)],
                osem.at[s])

        def compute(s):
            @plsc.parallel_loop(0, chunk // lanes, 1, unroll=2)
            def kbody(k):
                ivec = idx_v[pl.ds((s * chunk + k * lanes), lanes)]
                for u in range(lanes):
                    off = ivec[u] * d
                    obase = (s * chunk + k * lanes + u) * d
                    for half in range(d // lanes):
                        rows_v[pl.ds(obase + half * lanes, lanes)] = (
                            table_v[pl.ds(off + half * lanes, lanes)])

        idx_copy(0, 0).start()

        def chunk_body(i, carry):
            s = lax.rem(i, _NBUF)

            @pl.when(i + 1 < n_chunks)
            def _():
                idx_copy(i + 1, lax.rem(i + 1, _NBUF)).start()

            idx_copy(i, s).wait()

            @pl.when(i >= _NBUF)
            def _():
                out_copy(i - _NBUF, s).wait()

            compute(s)
            out_copy(i, s).start()
            return carry

        lax.fori_loop(0, n_chunks, chunk_body, 0)
        for j in range(_NBUF):
            i = n_chunks - _NBUF + j
            out_copy(i, i % _NBUF).wait()

    out = run(idx, embed.reshape(v_rows * d))
    return out.reshape(b, t, c * h * d)


# speedup vs baseline: 7.2932x; 1.0227x over previous
import functools

import jax
import jax.numpy as jnp
from jax import lax
from jax.experimental import pallas as pl
from jax.experimental.pallas import tpu as pltpu
from jax.experimental.pallas import tpu_sc as plsc

_NBUF = 3
_UNROLL = 8


def kernel(spikes, embed):
    b, t, c, h = spikes.shape
    v_rows, d = embed.shape
    n = b * t * c * h
    idx = spikes.reshape(n)

    info = plsc.get_sparse_core_info()
    nw = info.num_cores * info.num_subcores
    lanes = info.num_lanes
    per_w = n // nw
    chunk = 1024
    n_chunks = per_w // chunk

    mesh = plsc.VectorSubcoreMesh(core_axis_name="c", subcore_axis_name="s")

    @functools.partial(
        pl.kernel,
        mesh=mesh,
        out_type=jax.ShapeDtypeStruct((n * d,), jnp.float32),
        scratch_types=[
            pltpu.VMEM((v_rows * d,), jnp.float32),
            pltpu.VMEM((_NBUF * chunk,), jnp.int32),
            pltpu.VMEM((_NBUF * chunk * d,), jnp.float32),
            pltpu.SemaphoreType.DMA((_NBUF,)),
            pltpu.SemaphoreType.DMA((_NBUF,)),
        ],
    )
    def run(idx_hbm, table_hbm, out_hbm, table_v, idx_v, rows_v, isem, osem):
        wid = lax.axis_index("s") * info.num_cores + lax.axis_index("c")
        base = wid * per_w

        pltpu.sync_copy(table_hbm, table_v)

        def idx_copy(i, s):
            return pltpu.make_async_copy(
                idx_hbm.at[pl.ds(base + i * chunk, chunk)],
                idx_v.at[pl.ds(s * chunk, chunk)], isem.at[s])

        def out_copy(i, s):
            return pltpu.make_async_copy(
                rows_v.at[pl.ds(s * chunk * d, chunk * d)],
                out_hbm.at[pl.ds((base + i * chunk) * d, chunk * d)],
                osem.at[s])

        def compute(s):
            @plsc.parallel_loop(0, chunk // lanes, 1, unroll=4)
            def kbody(k):
                ivec = idx_v[pl.ds((s * chunk + k * lanes), lanes)]
                for u in range(lanes):
                    off = ivec[u] * d
                    obase = (s * chunk + k * lanes + u) * d
                    for half in range(d // lanes):
                        rows_v[pl.ds(obase + half * lanes, lanes)] = (
                            table_v[pl.ds(off + half * lanes, lanes)])

        idx_copy(0, 0).start()

        def chunk_body(i, carry):
            s = lax.rem(i, _NBUF)

            @pl.when(i + 1 < n_chunks)
            def _():
                idx_copy(i + 1, lax.rem(i + 1, _NBUF)).start()

            idx_copy(i, s).wait()

            @pl.when(i >= _NBUF)
            def _():
                out_copy(i - _NBUF, s).wait()

            compute(s)
            out_copy(i, s).start()
            return carry

        lax.fori_loop(0, n_chunks, chunk_body, 0)
        for j in range(_NBUF):
            i = n_chunks - _NBUF + j
            out_copy(i, i % _NBUF).wait()

    out = run(idx, embed.reshape(v_rows * d))
    return out.reshape(b, t, c * h * d)
